# baseline (speedup 1.0000x reference)
import functools

import jax
import jax.numpy as jnp
from jax import lax
from jax.experimental import pallas as pl
from jax.experimental.pallas import tpu as pltpu
from jax.experimental.pallas import tpu_sc as plsc

_THR = 0.3
_INTERPRET = False
_B = 128
_G = _B // 16
_W = 16


def _iou_conflict(bx1, by1, bx2, by2, bar, tx1, ty1, tx2, ty2, tar):
    xx1 = jnp.maximum(bx1, tx1)
    yy1 = jnp.maximum(by1, ty1)
    xx2 = jnp.minimum(bx2, tx2)
    yy2 = jnp.minimum(by2, ty2)
    w = jnp.maximum(0.0, xx2 - xx1)
    h = jnp.maximum(0.0, yy2 - yy1)
    inter = w * h
    iou = inter / ((bar + tar) - inter + 1e-8)
    return iou > _THR


def _make_sc_nms(npad):
    nblk = npad // _B
    cap = ((npad // _W) + 31) & ~15
    f32, i32 = jnp.float32, jnp.int32
    mesh = plsc.VectorSubcoreMesh(
        core_axis_name="c", subcore_axis_name="s", num_cores=1
    )

    @functools.partial(
        pl.kernel,
        mesh=mesh,
        out_type=jax.ShapeDtypeStruct((npad,), jnp.float32),
        compiler_params=pltpu.CompilerParams(needs_layout_passes=False),
        interpret=_INTERPRET,
        scratch_types=[
            pltpu.VMEM((npad,), f32),
            pltpu.VMEM((npad,), f32),
            pltpu.VMEM((npad,), f32),
            pltpu.VMEM((npad,), f32),
            pltpu.VMEM((cap,), i32),
            pltpu.VMEM((_B,), f32),
            pltpu.VMEM((_B,), f32),
            pltpu.VMEM((_W, _B), f32),
            pltpu.VMEM_SHARED((_W, _B), f32),
            pltpu.VMEM_SHARED((_B,), f32),
        ],
    )
    def sc_nms(x1h, y1h, x2h, y2h, keep_h, vx1, vy1, vx2, vy2,
               surv, mymask, av, slots_l, slots_s, alive_s):
        wid = lax.axis_index("s")
        iota16 = lax.broadcasted_iota(i32, (16,), 0)
        zeros16 = jnp.zeros((16,), f32)

        pltpu.sync_copy(x1h, vx1)
        pltpu.sync_copy(y1h, vy1)
        pltpu.sync_copy(x2h, vx2)
        pltpu.sync_copy(y2h, vy2)

        def block_body(k, gcnt):
            base = k * _B
            mycnt = jnp.maximum(0, (gcnt - wid + (_W - 1)) // _W)

            tx1 = [vx1[pl.ds(base + g * 16, 16)] for g in range(_G)]
            ty1 = [vy1[pl.ds(base + g * 16, 16)] for g in range(_G)]
            tx2 = [vx2[pl.ds(base + g * 16, 16)] for g in range(_G)]
            ty2 = [vy2[pl.ds(base + g * 16, 16)] for g in range(_G)]
            tar = [(tx2[g] - tx1[g]) * (ty2[g] - ty1[g]) for g in range(_G)]

            def sbody(s, accs):
                iv = plsc.load_gather(surv, [jnp.full((16,), s, i32)])
                bx1 = plsc.load_gather(vx1, [iv])
                by1 = plsc.load_gather(vy1, [iv])
                bx2 = plsc.load_gather(vx2, [iv])
                by2 = plsc.load_gather(vy2, [iv])
                bar = (bx2 - bx1) * (by2 - by1)
                out = []
                for g in range(_G):
                    conf = _iou_conflict(bx1, by1, bx2, by2, bar,
                                         tx1[g], ty1[g], tx2[g],
                                         ty2[g], tar[g])
                    out.append(jnp.where(conf, 1.0, accs[g]))
                return tuple(out)

            accs = lax.fori_loop(0, mycnt, sbody, (zeros16,) * _G)
            for g in range(_G):
                mymask[pl.ds(g * 16, 16)] = accs[g]

            pltpu.sync_copy(mymask, slots_s.at[wid])
            plsc.subcore_barrier()

            @pl.when(wid == 0)
            def _resolve():
                pltpu.sync_copy(slots_s, slots_l)
                for g in range(_G):
                    acc = zeros16
                    for w_ in range(_W):
                        acc = jnp.maximum(acc, slots_l[w_, pl.ds(g * 16, 16)])
                    av[pl.ds(g * 16, 16)] = 1.0 - acc

                def rbody(i, _):
                    a_i = plsc.load_gather(av, [jnp.full((16,), i, i32)])[0]

                    @pl.when(a_i > 0.5)
                    def _():
                        giv = jnp.full((16,), base + i, i32)
                        bx1 = plsc.load_gather(vx1, [giv])
                        by1 = plsc.load_gather(vy1, [giv])
                        bx2 = plsc.load_gather(vx2, [giv])
                        by2 = plsc.load_gather(vy2, [giv])
                        bar = (bx2 - bx1) * (by2 - by1)

                        def gbody(g, _2):
                            toff = base + g * 16
                            tx1 = vx1[pl.ds(toff, 16)]
                            ty1 = vy1[pl.ds(toff, 16)]
                            tx2 = vx2[pl.ds(toff, 16)]
                            ty2 = vy2[pl.ds(toff, 16)]
                            tar = (tx2 - tx1) * (ty2 - ty1)
                            conf = _iou_conflict(bx1, by1, bx2, by2, bar,
                                                 tx1, ty1, tx2, ty2, tar)
                            conf = conf & ((g * 16 + iota16) > i)
                            cur = av[pl.ds(g * 16, 16)]
                            av[pl.ds(g * 16, 16)] = jnp.where(conf, 0.0, cur)
                            return 0

                        lax.fori_loop(i // 16, _G, gbody, 0)
                    return 0

                lax.fori_loop(0, _B, rbody, 0)
                pltpu.sync_copy(av, alive_s)
                pltpu.sync_copy(av, keep_h.at[pl.ds(base, _B)])

            plsc.subcore_barrier()

            pltpu.sync_copy(alive_s, av)

            def abody(g, gc):
                a = av[pl.ds(g * 16, 16)]
                ai = a.astype(i32)
                inc = jnp.cumsum(ai)
                ordv = gc + (inc - ai)
                mine = (a > 0.5) & ((ordv & (_W - 1)) == wid)
                pos = lax.shift_right_logical(ordv, 4)
                gidx = base + g * 16 + iota16
                plsc.store_scatter(surv, [pos], gidx, mask=mine)
                return gc + jnp.sum(ai)

            gcnt = lax.fori_loop(0, _G, abody, gcnt)
            return gcnt

        lax.fori_loop(0, nblk, block_body, jnp.int32(0))

    return sc_nms


@jax.jit
def kernel(boxes, scores):
    n = boxes.shape[0]
    order = jnp.argsort(-scores)
    b = jnp.take(boxes, order, axis=0)
    s = jnp.take(scores, order)

    nblk = (n + _B - 1) // _B
    npad = nblk * _B
    bp = jnp.pad(b, ((0, npad - n), (0, 0)))
    keep = _make_sc_nms(npad)(
        bp[:, 0], bp[:, 1], bp[:, 2], bp[:, 3]
    )[:n]
    return jnp.concatenate([b * keep[:, None], (s * keep)[:, None]], axis=1)


# speedup vs baseline: 2.0341x; 2.0341x over previous
import functools

import jax
import jax.numpy as jnp
from jax import lax
from jax.experimental import pallas as pl
from jax.experimental.pallas import tpu as pltpu
from jax.experimental.pallas import tpu_sc as plsc

_THR = 0.3
_INTERPRET = False
_B = 128
_G = _B // 16
_W = 16
_WS = _W - 1


def _iou_conflict(bx1, by1, bx2, by2, bar, tx1, ty1, tx2, ty2, tar):
    xx1 = jnp.maximum(bx1, tx1)
    yy1 = jnp.maximum(by1, ty1)
    xx2 = jnp.minimum(bx2, tx2)
    yy2 = jnp.minimum(by2, ty2)
    w = jnp.maximum(0.0, xx2 - xx1)
    h = jnp.maximum(0.0, yy2 - yy1)
    inter = w * h
    iou = inter / ((bar + tar) - inter + 1e-8)
    return iou > _THR


def _make_sc_nms(npad):
    nblk = npad // _B
    cap = ((npad // _WS) + 31) & ~15
    f32, i32 = jnp.float32, jnp.int32
    mesh = plsc.VectorSubcoreMesh(
        core_axis_name="c", subcore_axis_name="s", num_cores=1
    )

    @functools.partial(
        pl.kernel,
        mesh=mesh,
        out_type=jax.ShapeDtypeStruct((npad,), jnp.float32),
        compiler_params=pltpu.CompilerParams(needs_layout_passes=False),
        interpret=_INTERPRET,
        scratch_types=[
            pltpu.VMEM((npad,), f32),
            pltpu.VMEM((npad,), f32),
            pltpu.VMEM((npad,), f32),
            pltpu.VMEM((npad,), f32),
            pltpu.VMEM((cap,), i32),
            pltpu.VMEM((_B,), f32),
            pltpu.VMEM((_B,), f32),
            pltpu.VMEM((_W, _B), f32),
            pltpu.VMEM_SHARED((_W, _B), f32),
            pltpu.VMEM_SHARED((_B,), f32),
        ],
    )
    def sc_nms(x1h, y1h, x2h, y2h, keep_h, vx1, vy1, vx2, vy2,
               surv, mymask, av, slots_l, slots_s, alive_s):
        wid = lax.axis_index("s")
        iota16 = lax.broadcasted_iota(i32, (16,), 0)
        zeros16 = jnp.zeros((16,), f32)

        pltpu.sync_copy(x1h, vx1)
        pltpu.sync_copy(y1h, vy1)
        pltpu.sync_copy(x2h, vx2)
        pltpu.sync_copy(y2h, vy2)

        def cross_range(tbase, lo, hi):
            for half in range(2):
                toff = tbase + half * 64
                tx1 = [vx1[pl.ds(toff + g * 16, 16)] for g in range(4)]
                ty1 = [vy1[pl.ds(toff + g * 16, 16)] for g in range(4)]
                tx2 = [vx2[pl.ds(toff + g * 16, 16)] for g in range(4)]
                ty2 = [vy2[pl.ds(toff + g * 16, 16)] for g in range(4)]
                tar = [(tx2[g] - tx1[g]) * (ty2[g] - ty1[g]) for g in range(4)]

                def sbody(s, accs, _tx1=tx1, _ty1=ty1, _tx2=tx2, _ty2=ty2,
                          _tar=tar):
                    iv = plsc.load_gather(surv, [jnp.full((16,), s, i32)])
                    bx1 = plsc.load_gather(vx1, [iv])
                    by1 = plsc.load_gather(vy1, [iv])
                    bx2 = plsc.load_gather(vx2, [iv])
                    by2 = plsc.load_gather(vy2, [iv])
                    bar = (bx2 - bx1) * (by2 - by1)
                    out = []
                    for g in range(4):
                        conf = _iou_conflict(bx1, by1, bx2, by2, bar,
                                             _tx1[g], _ty1[g], _tx2[g],
                                             _ty2[g], _tar[g])
                        out.append(jnp.where(conf, 1.0, accs[g]))
                    return tuple(out)

                init = tuple(
                    mymask[pl.ds(half * 64 + g * 16, 16)] for g in range(4)
                )
                accs = lax.fori_loop(lo, hi, sbody, init)
                for g in range(4):
                    mymask[pl.ds(half * 64 + g * 16, 16)] = accs[g]

        def my_shard_count(gc):
            return jnp.maximum(0, (gc - (wid - 1) + (_WS - 1)) // _WS)

        @pl.when(wid == 0)
        def _zero_slot0():
            for g in range(_G):
                mymask[pl.ds(g * 16, 16)] = zeros16
            pltpu.sync_copy(mymask, slots_s.at[0])

        @pl.when(wid > 0)
        def _zero_mask():
            for g in range(_G):
                mymask[pl.ds(g * 16, 16)] = zeros16

        plsc.subcore_barrier()

        def loop(k, gcnt):
            base = k * _B

            omy = my_shard_count(gcnt)

            def abody(g, gc):
                a = av[pl.ds(g * 16, 16)]
                ai = a.astype(i32)
                inc = jnp.cumsum(ai)
                ordv = gc + (inc - ai)
                mine = (a > 0.5) & ((ordv % _WS) == (wid - 1))
                pos = ordv // _WS
                gidx = (base - _B) + g * 16 + iota16
                plsc.store_scatter(surv, [pos], gidx, mask=mine)
                return gc + jnp.sum(ai)

            @pl.when((wid > 0) & (k > 0))
            def _():
                lax.fori_loop(0, _G, abody, gcnt)

            def cbody(g, gc):
                return gc + jnp.sum(av[pl.ds(g * 16, 16)].astype(i32))

            gcnt2 = lax.cond(
                k > 0,
                lambda: lax.fori_loop(0, _G, cbody, gcnt),
                lambda: gcnt,
            )
            nmy = my_shard_count(gcnt2)

            @pl.when((wid > 0) & (k > 0))
            def _():
                cross_range(base, omy, nmy)

            @pl.when(wid > 0)
            def _():
                pltpu.sync_copy(mymask, slots_s.at[wid])

            plsc.subcore_barrier()

            @pl.when(wid == 0)
            def _resolve():
                pltpu.sync_copy(slots_s, slots_l)
                for g in range(_G):
                    acc = zeros16
                    for w_ in range(_W):
                        acc = jnp.maximum(acc, slots_l[w_, pl.ds(g * 16, 16)])
                    av[pl.ds(g * 16, 16)] = 1.0 - acc

                def rbody(i, _2):
                    a_i = plsc.load_gather(av, [jnp.full((16,), i, i32)])[0]

                    @pl.when(a_i > 0.5)
                    def _3():
                        giv = jnp.full((16,), base + i, i32)
                        bx1 = plsc.load_gather(vx1, [giv])
                        by1 = plsc.load_gather(vy1, [giv])
                        bx2 = plsc.load_gather(vx2, [giv])
                        by2 = plsc.load_gather(vy2, [giv])
                        bar = (bx2 - bx1) * (by2 - by1)

                        def gbody(g, _4):
                            toff = base + g * 16
                            tx1 = vx1[pl.ds(toff, 16)]
                            ty1 = vy1[pl.ds(toff, 16)]
                            tx2 = vx2[pl.ds(toff, 16)]
                            ty2 = vy2[pl.ds(toff, 16)]
                            tar = (tx2 - tx1) * (ty2 - ty1)
                            conf = _iou_conflict(bx1, by1, bx2, by2, bar,
                                                 tx1, ty1, tx2, ty2, tar)
                            conf = conf & ((g * 16 + iota16) > i)
                            cur = av[pl.ds(g * 16, 16)]
                            av[pl.ds(g * 16, 16)] = jnp.where(conf, 0.0, cur)
                            return 0

                        lax.fori_loop(i // 16, _G, gbody, 0)
                    return 0

                lax.fori_loop(0, _B, rbody, 0)
                pltpu.sync_copy(av, alive_s)
                pltpu.sync_copy(av, keep_h.at[pl.ds(base, _B)])

            @pl.when((wid > 0) & (k + 1 < nblk))
            def _():
                for g in range(_G):
                    mymask[pl.ds(g * 16, 16)] = zeros16
                cross_range(base + _B, 0, nmy)

            plsc.subcore_barrier()

            pltpu.sync_copy(alive_s, av)
            return gcnt2

        lax.fori_loop(0, nblk, loop, jnp.int32(0))

    return sc_nms


@jax.jit
def kernel(boxes, scores):
    n = boxes.shape[0]
    order = jnp.argsort(-scores)
    b = jnp.take(boxes, order, axis=0)
    s = jnp.take(scores, order)

    nblk = (n + _B - 1) // _B
    npad = nblk * _B
    bp = jnp.pad(b, ((0, npad - n), (0, 0)))
    keep = _make_sc_nms(npad)(
        bp[:, 0], bp[:, 1], bp[:, 2], bp[:, 3]
    )[:n]
    return jnp.concatenate([b * keep[:, None], (s * keep)[:, None]], axis=1)
